# baseline (speedup 1.0000x reference)
import functools

import jax
import jax.numpy as jnp
from jax import lax
from jax.experimental import pallas as pl
from jax.experimental.pallas import tpu as pltpu
from jax.experimental.pallas import tpu_sc as plsc

N = 50000
E = 800000
NHID = 96
NOUT = 64
NGRU = 64
NHN = 64

N_PAD = 50176
NSLICE = N_PAD // 16
E_PAD = 819200
EROWS = E_PAD // 128
ROWS_PER_TILE = EROWS // 16
ROW_TILE = 512
GRID_N = N_PAD // ROW_TILE

_mesh = lambda: plsc.VectorSubcoreMesh(core_axis_name="c", subcore_axis_name="s")


def _make_spmm(C, day_stride):
    NB = 50

    def body(xf, rows_h, cols_h, out_h,
             colA, rowA, cvA, valA, colB, rowB, cvB, valB, zv, acc,
             sgA, sgB, ssA, ssB):
        cid = lax.axis_index("c")
        sid = lax.axis_index("s")

        def zinit(i, _):
            zv[i, :] = jnp.zeros((16,), jnp.float32)
            return 0

        lax.fori_loop(0, 392, zinit, 0)
        for i in range(8):
            pltpu.sync_copy(zv, acc.at[pl.ds(sid * NSLICE + i * 392, 392)])
        plsc.subcore_barrier()

        def mkhalf(colv, rowv, cv2, val, sg, ss, c, d):
            off = c + d * day_stride

            def load_idx(b):
                r0 = sid * ROWS_PER_TILE + b * 8
                pltpu.sync_copy(cols_h.at[d].at[pl.ds(r0, 8)], colv)
                pltpu.sync_copy(rows_h.at[d].at[pl.ds(r0, 8)], rowv)
                for j in range(8):
                    for k in range(8):
                        s = colv[j, pl.ds(k * 16, 16)]
                        cv2[j, pl.ds(k * 16, 16)] = s * C + off

            def fire_g():
                for j in range(8):
                    pltpu.async_copy(xf.at[cv2.at[j]], val.at[j], sg)

            def wait_g():
                for j in range(8):
                    pltpu.make_async_copy(xf.at[cv2.at[j]], val.at[j],
                                          sg).wait()

            def fire_s():
                for j in range(8):
                    pltpu.async_copy(val.at[j], acc.at[rowv.at[j]], ss,
                                     add=True)

            def wait_s():
                for j in range(8):
                    pltpu.make_async_copy(val.at[j], acc.at[rowv.at[j]],
                                          ss).wait()

            return load_idx, fire_g, wait_g, fire_s, wait_s

        for c in range(C):
            @pl.when(cid == c % 2)
            def _(c=c):
                def day(d, _):
                    liA, fgA, wgA, fsA, wsA = mkhalf(colA, rowA, cvA, valA,
                                                     sgA, ssA, c, d)
                    liB, fgB, wgB, fsB, wsB = mkhalf(colB, rowB, cvB, valB,
                                                     sgB, ssB, c, d)
                    liA(0)
                    fgA()

                    def blk(i, _):
                        @pl.when(i > 0)
                        def _w():
                            wsB()

                        liB(2 * i + 1)
                        fgB()
                        wgA()
                        fsA()
                        wsA()

                        @pl.when(i < NB // 2 - 1)
                        def _p():
                            liA(2 * i + 2)
                            fgA()

                        wgB()
                        fsB()
                        return 0

                    lax.fori_loop(0, NB // 2, blk, 0)
                    wsB()
                    plsc.subcore_barrier()
                    pltpu.sync_copy(acc.at[pl.ds(sid * NSLICE, NSLICE)],
                                    out_h.at[d].at[pl.ds(sid * NSLICE, NSLICE),
                                                   pl.ds(c * 16, 16)])
                    for i in range(8):
                        pltpu.sync_copy(
                            zv, acc.at[pl.ds(sid * NSLICE + i * 392, 392)])
                    plsc.subcore_barrier()
                    return 0

                lax.fori_loop(0, 7, day, 0)

    return pl.kernel(
        body,
        out_type=jax.ShapeDtypeStruct((7, N_PAD, C * 16), jnp.float32),
        mesh=_mesh(),
        compiler_params=pltpu.CompilerParams(use_tc_tiling_on_sc=False),
        scratch_types=[
            pltpu.VMEM((8, 128), jnp.int32),
            pltpu.VMEM((8, 128), jnp.int32),
            pltpu.VMEM((8, 128), jnp.int32),
            pltpu.VMEM((8, 128, 16), jnp.float32),
            pltpu.VMEM((8, 128), jnp.int32),
            pltpu.VMEM((8, 128), jnp.int32),
            pltpu.VMEM((8, 128), jnp.int32),
            pltpu.VMEM((8, 128, 16), jnp.float32),
            pltpu.VMEM((392, 16), jnp.float32),
            pltpu.VMEM_SHARED((N_PAD, 16), jnp.float32),
            pltpu.SemaphoreType.DMA,
            pltpu.SemaphoreType.DMA,
            pltpu.SemaphoreType.DMA,
            pltpu.SemaphoreType.DMA,
        ],
    )


_spmm6 = _make_spmm(6, 0)
_spmm4 = _make_spmm(4, N_PAD * 4)


def _attn_body(embf, rows_h, cols_h, aT_h, bT_h, out_h, deg_h,
               colA, rowA, cvA, avA, bvA, wvA, mvA, ebA,
               colB, rowB, cvB, avB, bvB, wvB, mvB, ebB,
               zv, zv1, aT_s, bT_s, acc, dacc,
               sgA, sgB, stA, stB, ssA, ssB, sdA, sdB):
    cid = lax.axis_index("c")
    sid = lax.axis_index("s")

    @pl.when(sid == 0)
    def _():
        pltpu.sync_copy(aT_h, aT_s)
        pltpu.sync_copy(bT_h, bT_s)

    def zinit(i, _):
        zv[i, pl.ds(0, 16)] = jnp.zeros((16,), jnp.float32)
        zv[i, pl.ds(16, 16)] = jnp.zeros((16,), jnp.float32)
        return 0

    lax.fori_loop(0, 16, zinit, 0)

    def zinit1(i, _):
        zv1[pl.ds(i * 16, 16)] = jnp.zeros((16,), jnp.float32)
        return 0

    lax.fori_loop(0, 28, zinit1, 0)
    for i in range(196):
        pltpu.sync_copy(zv, acc.at[pl.ds(sid * NSLICE + i * 16, 16)])
    for i in range(7):
        pltpu.sync_copy(zv1, dacc.at[pl.ds(sid * NSLICE + i * 448, 448)])
    plsc.subcore_barrier()

    for c in range(2):
        @pl.when(cid == c)
        def _(c=c):
            def mkhalf(colv, rowv, cv2, av, bv, wv, mv, ebuf, sg, st, ss, sd):
                def li(b):
                    r0 = sid * ROWS_PER_TILE + b * 2
                    pltpu.sync_copy(cols_h.at[pl.ds(r0, 2)], colv)
                    pltpu.sync_copy(rows_h.at[pl.ds(r0, 2)], rowv)
                    for j in range(2):
                        for k in range(8):
                            s = colv[j, pl.ds(k * 16, 16)]
                            cv2[j, pl.ds(k * 16, 16)] = s * 2 + c

                def fg():
                    for j in range(2):
                        pltpu.async_copy(aT_s.at[rowv.at[j]], av.at[j], st)
                        pltpu.async_copy(bT_s.at[colv.at[j]], bv.at[j], st)
                        pltpu.async_copy(embf.at[cv2.at[j]], ebuf.at[j], sg)

                def wg():
                    for j in range(2):
                        pltpu.make_async_copy(aT_s.at[rowv.at[j]], av.at[j],
                                              st).wait()
                        pltpu.make_async_copy(bT_s.at[colv.at[j]], bv.at[j],
                                              st).wait()
                        pltpu.make_async_copy(embf.at[cv2.at[j]], ebuf.at[j],
                                              sg).wait()

                def compute():
                    for j in range(2):
                        for k in range(8):
                            r16 = rowv[j, pl.ds(k * 16, 16)]
                            c16 = colv[j, pl.ds(k * 16, 16)]
                            a16 = av[j, pl.ds(k * 16, 16)]
                            b16 = bv[j, pl.ds(k * 16, 16)]
                            w = 1.0 / (1.0 + jnp.exp(-(a16 + b16)))
                            m = r16 != c16
                            w = jnp.where(m, w, jnp.zeros((16,), jnp.float32))
                            wv[j, pl.ds(k * 16, 16)] = w
                            if c == 0:
                                mv[j, pl.ds(k * 16, 16)] = jnp.where(
                                    m, jnp.ones((16,), jnp.float32),
                                    jnp.zeros((16,), jnp.float32))

                    def scale(kk, _):
                        for j in range(2):
                            w16 = wv[j, pl.ds(kk * 16, 16)]
                            for l in range(16):
                                s16 = jnp.full((16,), w16[l], jnp.float32)
                                t0 = ebuf[j, kk * 16 + l, pl.ds(0, 16)]
                                ebuf[j, kk * 16 + l, pl.ds(0, 16)] = t0 * s16
                                t1 = ebuf[j, kk * 16 + l, pl.ds(16, 16)]
                                ebuf[j, kk * 16 + l, pl.ds(16, 16)] = t1 * s16
                        return 0

                    lax.fori_loop(0, 8, scale, 0)

                def fs():
                    for j in range(2):
                        pltpu.async_copy(ebuf.at[j], acc.at[rowv.at[j]], ss,
                                         add=True)
                        if c == 0:
                            pltpu.async_copy(mv.at[j], dacc.at[rowv.at[j]],
                                             sd, add=True)

                def ws():
                    for j in range(2):
                        pltpu.make_async_copy(ebuf.at[j], acc.at[rowv.at[j]],
                                              ss).wait()
                        if c == 0:
                            pltpu.make_async_copy(mv.at[j],
                                                  dacc.at[rowv.at[j]],
                                                  sd).wait()

                return li, fg, wg, compute, fs, ws

            liA, fgA, wgA, cpA, fsA, wsA = mkhalf(colA, rowA, cvA, avA, bvA,
                                                  wvA, mvA, ebA, sgA, stA,
                                                  ssA, sdA)
            liB, fgB, wgB, cpB, fsB, wsB = mkhalf(colB, rowB, cvB, avB, bvB,
                                                  wvB, mvB, ebB, sgB, stB,
                                                  ssB, sdB)
            liA(0)
            fgA()

            def blk(i, _):
                @pl.when(i > 0)
                def _w():
                    wsB()

                liB(2 * i + 1)
                fgB()
                wgA()
                cpA()
                fsA()
                wsA()

                @pl.when(i < 99)
                def _p():
                    liA(2 * i + 2)
                    fgA()

                wgB()
                cpB()
                fsB()
                return 0

            lax.fori_loop(0, 100, blk, 0)
            wsB()
            plsc.subcore_barrier()
            pltpu.sync_copy(acc.at[pl.ds(sid * NSLICE, NSLICE)],
                            out_h.at[pl.ds(sid * NSLICE, NSLICE),
                                     pl.ds(c * 32, 32)])
            if c == 0:
                pltpu.sync_copy(dacc.at[pl.ds(sid * NSLICE, NSLICE)],
                                deg_h.at[pl.ds(sid * NSLICE, NSLICE)])


_attn = pl.kernel(
    _attn_body,
    out_type=(jax.ShapeDtypeStruct((N_PAD, NOUT), jnp.float32),
              jax.ShapeDtypeStruct((N_PAD,), jnp.float32)),
    mesh=_mesh(),
    compiler_params=pltpu.CompilerParams(use_tc_tiling_on_sc=False),
    scratch_types=(
        [pltpu.VMEM((2, 128), jnp.int32)] * 3
        + [pltpu.VMEM((2, 128), jnp.float32)] * 4
        + [pltpu.VMEM((2, 128, 32), jnp.float32)]
        + [pltpu.VMEM((2, 128), jnp.int32)] * 3
        + [pltpu.VMEM((2, 128), jnp.float32)] * 4
        + [pltpu.VMEM((2, 128, 32), jnp.float32)]
        + [
            pltpu.VMEM((16, 32), jnp.float32),
            pltpu.VMEM((448,), jnp.float32),
            pltpu.VMEM_SHARED((N_PAD,), jnp.float32),
            pltpu.VMEM_SHARED((N_PAD,), jnp.float32),
            pltpu.VMEM_SHARED((N_PAD, 32), jnp.float32),
            pltpu.VMEM_SHARED((N_PAD,), jnp.float32),
        ]
        + [pltpu.SemaphoreType.DMA] * 8
    ),
)


def _layer_a_body(x_ref, b0_ref, w1_ref, o_ref):
    x = jnp.maximum(x_ref[...] + b0_ref[...][0], 0.0)
    o_ref[...] = jnp.dot(x, w1_ref[...], preferred_element_type=jnp.float32)


def _layer_a(x, b0, W1):
    rows = x.shape[0]
    return pl.pallas_call(
        _layer_a_body,
        grid=(rows // ROW_TILE,),
        in_specs=[
            pl.BlockSpec((ROW_TILE, NHID), lambda i: (i, 0)),
            pl.BlockSpec((1, NHID), lambda i: (0, 0)),
            pl.BlockSpec((NHID, NOUT), lambda i: (0, 0)),
        ],
        out_specs=pl.BlockSpec((ROW_TILE, NOUT), lambda i: (i, 0)),
        out_shape=jax.ShapeDtypeStruct((rows, NOUT), jnp.float32),
    )(x, b0.reshape(1, NHID), W1)


def _gru_body(seq_ref, ws_ref, bs_ref, bn_ref, wa_ref, emb_ref, ab_ref):
    h = jnp.zeros((ROW_TILE, NGRU), jnp.float32)
    for t in range(7):
        x = seq_ref[t]
        r = jax.nn.sigmoid(
            jnp.dot(x, ws_ref[0], preferred_element_type=jnp.float32)
            + jnp.dot(h, ws_ref[3], preferred_element_type=jnp.float32)
            + bs_ref[...][0][None, :])
        z = jax.nn.sigmoid(
            jnp.dot(x, ws_ref[1], preferred_element_type=jnp.float32)
            + jnp.dot(h, ws_ref[4], preferred_element_type=jnp.float32)
            + bs_ref[...][1][None, :])
        hn = (jnp.dot(h, ws_ref[5], preferred_element_type=jnp.float32)
              + bs_ref[...][3][None, :])
        inn = (jnp.dot(x, ws_ref[2], preferred_element_type=jnp.float32)
               + bs_ref[...][2][None, :])
        n = jnp.tanh(inn + r * hn)
        h = (1.0 - z) * n + z * h
    emb = h * bn_ref[...][0][None, :] + bn_ref[...][1][None, :]
    emb_ref[...] = emb
    ab_ref[...] = jnp.dot(emb, wa_ref[...], preferred_element_type=jnp.float32)


def _gru(seq, ws, bs, bn2, wa128):
    return pl.pallas_call(
        _gru_body,
        grid=(GRID_N,),
        in_specs=[
            pl.BlockSpec((7, ROW_TILE, NGRU), lambda i: (0, i, 0)),
            pl.BlockSpec((6, NGRU, NGRU), lambda i: (0, 0, 0)),
            pl.BlockSpec((4, NGRU), lambda i: (0, 0)),
            pl.BlockSpec((2, NGRU), lambda i: (0, 0)),
            pl.BlockSpec((NGRU, 128), lambda i: (0, 0)),
        ],
        out_specs=[
            pl.BlockSpec((ROW_TILE, NGRU), lambda i: (i, 0)),
            pl.BlockSpec((ROW_TILE, 128), lambda i: (i, 0)),
        ],
        out_shape=[
            jax.ShapeDtypeStruct((N_PAD, NGRU), jnp.float32),
            jax.ShapeDtypeStruct((N_PAD, 128), jnp.float32),
        ],
    )(seq, ws, bs, bn2, wa128)


def _head_body(emb_ref, ngh_ref, wp1a_ref, wp1b_ref, bp1_ref, wp2_ref,
               bp2_ref, out_ref):
    hid = jnp.maximum(
        jnp.dot(emb_ref[...], wp1a_ref[...], preferred_element_type=jnp.float32)
        + jnp.dot(ngh_ref[...], wp1b_ref[...], preferred_element_type=jnp.float32)
        + bp1_ref[...][0],
        0.0,
    )
    logits = (jnp.dot(hid, wp2_ref[...], preferred_element_type=jnp.float32)
              + bp2_ref[...][0])
    m = jnp.max(logits, axis=1, keepdims=True)
    lse = m + jnp.log(jnp.sum(jnp.exp(logits - m), axis=1, keepdims=True))
    out_ref[...] = logits - lse


def _head(emb, ngh, wp1a, wp1b, bp1, wp2pad, bp2pad):
    return pl.pallas_call(
        _head_body,
        grid=(GRID_N,),
        in_specs=[
            pl.BlockSpec((ROW_TILE, NGRU), lambda i: (i, 0)),
            pl.BlockSpec((ROW_TILE, NGRU), lambda i: (i, 0)),
            pl.BlockSpec((NGRU, NHN), lambda i: (0, 0)),
            pl.BlockSpec((NGRU, NHN), lambda i: (0, 0)),
            pl.BlockSpec((1, NHN), lambda i: (0, 0)),
            pl.BlockSpec((NHN, 128), lambda i: (0, 0)),
            pl.BlockSpec((1, 128), lambda i: (0, 0)),
        ],
        out_specs=pl.BlockSpec((ROW_TILE, 128), lambda i: (i, 0)),
        out_shape=jax.ShapeDtypeStruct((N_PAD, 128), jnp.float32),
    )(emb, ngh, wp1a, wp1b, bp1.reshape(1, NHN), wp2pad, bp2pad)


def kernel(adjs, start_day, end_day, W0, b0, W1, b1, W_ih, W_hh, b_ih, b_hh,
           bn_g, bn_b, Wa, ba, Wp1, bp1, Wp2, bp2):
    npad = E_PAD - E
    it = jnp.arange(npad, dtype=jnp.int32)
    pad_rows = N + (it % (N_PAD - N))
    pad_cols = (it * 9973) % N
    pad2 = jnp.stack([pad_rows, pad_cols])
    adjp = jnp.concatenate(
        [adjs.astype(jnp.int32),
         jnp.broadcast_to(pad2[None], (adjs.shape[0], 2, npad))], axis=2)
    adjp = adjp.reshape(adjs.shape[0], 2, EROWS, 128)

    W0p = jnp.zeros((N_PAD, NHID), jnp.float32).at[:N].set(W0)
    W0f = W0p.reshape(N_PAD * 6, 16)

    sl = lax.dynamic_slice_in_dim(adjp, start_day, 7, axis=0)
    rows_all, cols_all = sl[:, 0], sl[:, 1]
    x0_all = _spmm6(W0f, rows_all, cols_all)
    h1_all = _layer_a(x0_all.reshape(7 * N_PAD, NHID), b0, W1)
    x1_all = _spmm4(h1_all.reshape(7 * N_PAD * 4, 16),
                    rows_all, cols_all)

    b_ih2 = b_ih + W_ih @ b1
    ws = jnp.stack([W_ih[0:64].T, W_ih[64:128].T, W_ih[128:192].T,
                    W_hh[0:64].T, W_hh[64:128].T, W_hh[128:192].T])
    bs = jnp.stack([b_ih2[0:64] + b_hh[0:64],
                    b_ih2[64:128] + b_hh[64:128],
                    b_ih2[128:192],
                    b_hh[128:192]])
    bn2 = jnp.stack([bn_g / jnp.sqrt(1.0 + 1e-5), bn_b])
    wa128 = jnp.zeros((NGRU, 128), jnp.float32)
    wa128 = wa128.at[:, 0].set(Wa[0, :64]).at[:, 1].set(Wa[0, 64:])

    emb, ab = _gru(x1_all, ws, bs, bn2, wa128)
    aT = ab[:, 0] + ba[0]
    bT = ab[:, 1]

    ea = lax.dynamic_index_in_dim(adjp, end_day + 1, 0, keepdims=False)
    neigh2, deg = _attn(emb.reshape(N_PAD * 2, 32), ea[0], ea[1], aT, bT)
    diag = jnp.where(deg != 0, 1.0 / jnp.where(deg != 0, deg, 1.0), 1.0)
    neigh = neigh2 * diag[:, None]

    wp2pad = jnp.zeros((NHN, 128), jnp.float32).at[:, :2].set(Wp2.T)
    bp2pad = jnp.full((1, 128), -1e30, jnp.float32).at[0, :2].set(bp2)
    pred = _head(emb, neigh, Wp1[:, :64].T, Wp1[:, 64:].T, bp1, wp2pad, bp2pad)
    return pred[:N, :2]


# speedup vs baseline: 1.0687x; 1.0687x over previous
import functools

import jax
import jax.numpy as jnp
from jax import lax
from jax.experimental import pallas as pl
from jax.experimental.pallas import tpu as pltpu
from jax.experimental.pallas import tpu_sc as plsc

N = 50000
E = 800000
NHID = 96
NOUT = 64
NGRU = 64
NHN = 64

N_PAD = 50176
NSLICE = N_PAD // 16
E_PAD = 819200
EROWS = E_PAD // 128
ROWS_PER_TILE = EROWS // 16
ROW_TILE = 512
GRID_N = N_PAD // ROW_TILE

_mesh = lambda: plsc.VectorSubcoreMesh(core_axis_name="c", subcore_axis_name="s")


def _make_spmm(C, day_stride):
    NB = 50

    def body(xf, rows_h, cols_h, out_h,
             colA, rowA, cvA, valA, colB, rowB, cvB, valB, zv, acc,
             sgA, sgB, ssA, ssB):
        cid = lax.axis_index("c")
        sid = lax.axis_index("s")

        def zinit(i, _):
            zv[i, :] = jnp.zeros((16,), jnp.float32)
            return 0

        lax.fori_loop(0, 392, zinit, 0)
        for i in range(8):
            pltpu.sync_copy(zv, acc.at[pl.ds(sid * NSLICE + i * 392, 392)])
        plsc.subcore_barrier()

        def mkhalf(colv, rowv, cv2, val, sg, ss, c):
            off = c

            def load_idx(b):
                r0 = sid * ROWS_PER_TILE + b * 8
                pltpu.sync_copy(cols_h.at[pl.ds(r0, 8)], colv)
                pltpu.sync_copy(rows_h.at[pl.ds(r0, 8)], rowv)
                for j in range(8):
                    for k in range(8):
                        s = colv[j, pl.ds(k * 16, 16)]
                        cv2[j, pl.ds(k * 16, 16)] = s * C + off

            def fire_g():
                for j in range(8):
                    pltpu.async_copy(xf.at[cv2.at[j]], val.at[j], sg)

            def wait_g():
                for j in range(8):
                    pltpu.make_async_copy(xf.at[cv2.at[j]], val.at[j],
                                          sg).wait()

            def fire_s():
                for j in range(8):
                    pltpu.async_copy(val.at[j], acc.at[rowv.at[j]], ss,
                                     add=True)

            def wait_s():
                for j in range(8):
                    pltpu.make_async_copy(val.at[j], acc.at[rowv.at[j]],
                                          ss).wait()

            return load_idx, fire_g, wait_g, fire_s, wait_s

        for c in range(C):
            @pl.when(cid == c % 2)
            def _(c=c):
                liA, fgA, wgA, fsA, wsA = mkhalf(colA, rowA, cvA, valA,
                                                 sgA, ssA, c)
                liB, fgB, wgB, fsB, wsB = mkhalf(colB, rowB, cvB, valB,
                                                 sgB, ssB, c)
                liA(0)
                fgA()

                def blk(i, _):
                    @pl.when(i > 0)
                    def _w():
                        wsB()

                    liB(2 * i + 1)
                    fgB()
                    wgA()
                    fsA()
                    wsA()

                    @pl.when(i < NB // 2 - 1)
                    def _p():
                        liA(2 * i + 2)
                        fgA()

                    wgB()
                    fsB()
                    return 0

                lax.fori_loop(0, NB // 2, blk, 0)
                wsB()
                plsc.subcore_barrier()
                pltpu.sync_copy(acc.at[pl.ds(sid * NSLICE, NSLICE)],
                                out_h.at[pl.ds(sid * NSLICE, NSLICE),
                                         pl.ds(c * 16, 16)])
                for i in range(8):
                    pltpu.sync_copy(
                        zv, acc.at[pl.ds(sid * NSLICE + i * 392, 392)])
                plsc.subcore_barrier()

    return pl.kernel(
        body,
        out_type=jax.ShapeDtypeStruct((N_PAD, C * 16), jnp.float32),
        mesh=_mesh(),
        compiler_params=pltpu.CompilerParams(use_tc_tiling_on_sc=False),
        scratch_types=[
            pltpu.VMEM((8, 128), jnp.int32),
            pltpu.VMEM((8, 128), jnp.int32),
            pltpu.VMEM((8, 128), jnp.int32),
            pltpu.VMEM((8, 128, 16), jnp.float32),
            pltpu.VMEM((8, 128), jnp.int32),
            pltpu.VMEM((8, 128), jnp.int32),
            pltpu.VMEM((8, 128), jnp.int32),
            pltpu.VMEM((8, 128, 16), jnp.float32),
            pltpu.VMEM((392, 16), jnp.float32),
            pltpu.VMEM_SHARED((N_PAD, 16), jnp.float32),
            pltpu.SemaphoreType.DMA,
            pltpu.SemaphoreType.DMA,
            pltpu.SemaphoreType.DMA,
            pltpu.SemaphoreType.DMA,
        ],
    )


_spmm6 = _make_spmm(6, 0)
_spmm4 = _make_spmm(4, 0)


def _attn_body(embf, rows_h, cols_h, aT_h, bT_h, out_h, deg_h,
               colA, rowA, cvA, avA, bvA, wvA, mvA, ebA,
               colB, rowB, cvB, avB, bvB, wvB, mvB, ebB,
               zv, zv1, aT_s, bT_s, acc, dacc,
               sgA, sgB, stA, stB, ssA, ssB, sdA, sdB):
    cid = lax.axis_index("c")
    sid = lax.axis_index("s")

    @pl.when(sid == 0)
    def _():
        pltpu.sync_copy(aT_h, aT_s)
        pltpu.sync_copy(bT_h, bT_s)

    def zinit(i, _):
        zv[i, pl.ds(0, 16)] = jnp.zeros((16,), jnp.float32)
        zv[i, pl.ds(16, 16)] = jnp.zeros((16,), jnp.float32)
        return 0

    lax.fori_loop(0, 16, zinit, 0)

    def zinit1(i, _):
        zv1[pl.ds(i * 16, 16)] = jnp.zeros((16,), jnp.float32)
        return 0

    lax.fori_loop(0, 28, zinit1, 0)
    for i in range(196):
        pltpu.sync_copy(zv, acc.at[pl.ds(sid * NSLICE + i * 16, 16)])
    for i in range(7):
        pltpu.sync_copy(zv1, dacc.at[pl.ds(sid * NSLICE + i * 448, 448)])
    plsc.subcore_barrier()

    for c in range(2):
        @pl.when(cid == c)
        def _(c=c):
            def mkhalf(colv, rowv, cv2, av, bv, wv, mv, ebuf, sg, st, ss, sd):
                def li(b):
                    r0 = sid * ROWS_PER_TILE + b * 2
                    pltpu.sync_copy(cols_h.at[pl.ds(r0, 2)], colv)
                    pltpu.sync_copy(rows_h.at[pl.ds(r0, 2)], rowv)
                    for j in range(2):
                        for k in range(8):
                            s = colv[j, pl.ds(k * 16, 16)]
                            cv2[j, pl.ds(k * 16, 16)] = s * 2 + c

                def fg():
                    for j in range(2):
                        pltpu.async_copy(aT_s.at[rowv.at[j]], av.at[j], st)
                        pltpu.async_copy(bT_s.at[colv.at[j]], bv.at[j], st)
                        pltpu.async_copy(embf.at[cv2.at[j]], ebuf.at[j], sg)

                def wg():
                    for j in range(2):
                        pltpu.make_async_copy(aT_s.at[rowv.at[j]], av.at[j],
                                              st).wait()
                        pltpu.make_async_copy(bT_s.at[colv.at[j]], bv.at[j],
                                              st).wait()
                        pltpu.make_async_copy(embf.at[cv2.at[j]], ebuf.at[j],
                                              sg).wait()

                def compute():
                    for j in range(2):
                        for k in range(8):
                            r16 = rowv[j, pl.ds(k * 16, 16)]
                            c16 = colv[j, pl.ds(k * 16, 16)]
                            a16 = av[j, pl.ds(k * 16, 16)]
                            b16 = bv[j, pl.ds(k * 16, 16)]
                            w = 1.0 / (1.0 + jnp.exp(-(a16 + b16)))
                            m = r16 != c16
                            w = jnp.where(m, w, jnp.zeros((16,), jnp.float32))
                            wv[j, pl.ds(k * 16, 16)] = w
                            if c == 0:
                                mv[j, pl.ds(k * 16, 16)] = jnp.where(
                                    m, jnp.ones((16,), jnp.float32),
                                    jnp.zeros((16,), jnp.float32))

                    def scale(kk, _):
                        for j in range(2):
                            w16 = wv[j, pl.ds(kk * 16, 16)]
                            for l in range(16):
                                s16 = jnp.full((16,), w16[l], jnp.float32)
                                t0 = ebuf[j, kk * 16 + l, pl.ds(0, 16)]
                                ebuf[j, kk * 16 + l, pl.ds(0, 16)] = t0 * s16
                                t1 = ebuf[j, kk * 16 + l, pl.ds(16, 16)]
                                ebuf[j, kk * 16 + l, pl.ds(16, 16)] = t1 * s16
                        return 0

                    lax.fori_loop(0, 8, scale, 0)

                def fs():
                    for j in range(2):
                        pltpu.async_copy(ebuf.at[j], acc.at[rowv.at[j]], ss,
                                         add=True)
                        if c == 0:
                            pltpu.async_copy(mv.at[j], dacc.at[rowv.at[j]],
                                             sd, add=True)

                def ws():
                    for j in range(2):
                        pltpu.make_async_copy(ebuf.at[j], acc.at[rowv.at[j]],
                                              ss).wait()
                        if c == 0:
                            pltpu.make_async_copy(mv.at[j],
                                                  dacc.at[rowv.at[j]],
                                                  sd).wait()

                return li, fg, wg, compute, fs, ws

            liA, fgA, wgA, cpA, fsA, wsA = mkhalf(colA, rowA, cvA, avA, bvA,
                                                  wvA, mvA, ebA, sgA, stA,
                                                  ssA, sdA)
            liB, fgB, wgB, cpB, fsB, wsB = mkhalf(colB, rowB, cvB, avB, bvB,
                                                  wvB, mvB, ebB, sgB, stB,
                                                  ssB, sdB)
            liA(0)
            fgA()

            def blk(i, _):
                @pl.when(i > 0)
                def _w():
                    wsB()

                liB(2 * i + 1)
                fgB()
                wgA()
                cpA()
                fsA()
                wsA()

                @pl.when(i < 99)
                def _p():
                    liA(2 * i + 2)
                    fgA()

                wgB()
                cpB()
                fsB()
                return 0

            lax.fori_loop(0, 100, blk, 0)
            wsB()
            plsc.subcore_barrier()
            pltpu.sync_copy(acc.at[pl.ds(sid * NSLICE, NSLICE)],
                            out_h.at[pl.ds(sid * NSLICE, NSLICE),
                                     pl.ds(c * 32, 32)])
            if c == 0:
                pltpu.sync_copy(dacc.at[pl.ds(sid * NSLICE, NSLICE)],
                                deg_h.at[pl.ds(sid * NSLICE, NSLICE)])


_attn = pl.kernel(
    _attn_body,
    out_type=(jax.ShapeDtypeStruct((N_PAD, NOUT), jnp.float32),
              jax.ShapeDtypeStruct((N_PAD,), jnp.float32)),
    mesh=_mesh(),
    compiler_params=pltpu.CompilerParams(use_tc_tiling_on_sc=False),
    scratch_types=(
        [pltpu.VMEM((2, 128), jnp.int32)] * 3
        + [pltpu.VMEM((2, 128), jnp.float32)] * 4
        + [pltpu.VMEM((2, 128, 32), jnp.float32)]
        + [pltpu.VMEM((2, 128), jnp.int32)] * 3
        + [pltpu.VMEM((2, 128), jnp.float32)] * 4
        + [pltpu.VMEM((2, 128, 32), jnp.float32)]
        + [
            pltpu.VMEM((16, 32), jnp.float32),
            pltpu.VMEM((448,), jnp.float32),
            pltpu.VMEM_SHARED((N_PAD,), jnp.float32),
            pltpu.VMEM_SHARED((N_PAD,), jnp.float32),
            pltpu.VMEM_SHARED((N_PAD, 32), jnp.float32),
            pltpu.VMEM_SHARED((N_PAD,), jnp.float32),
        ]
        + [pltpu.SemaphoreType.DMA] * 8
    ),
)


def _layer_a_body(x_ref, b0_ref, w1_ref, o_ref):
    x = jnp.maximum(x_ref[...] + b0_ref[...][0], 0.0)
    o_ref[...] = jnp.dot(x, w1_ref[...], preferred_element_type=jnp.float32)


def _layer_a(x, b0, W1):
    rows = x.shape[0]
    return pl.pallas_call(
        _layer_a_body,
        grid=(rows // ROW_TILE,),
        in_specs=[
            pl.BlockSpec((ROW_TILE, NHID), lambda i: (i, 0)),
            pl.BlockSpec((1, NHID), lambda i: (0, 0)),
            pl.BlockSpec((NHID, NOUT), lambda i: (0, 0)),
        ],
        out_specs=pl.BlockSpec((ROW_TILE, NOUT), lambda i: (i, 0)),
        out_shape=jax.ShapeDtypeStruct((rows, NOUT), jnp.float32),
    )(x, b0.reshape(1, NHID), W1)


def _gru_body(*refs):
    x_refs = refs[:7]
    ws_ref, bs_ref, bn_ref, wa_ref, emb_ref, ab_ref = refs[7:]
    h = jnp.zeros((ROW_TILE, NGRU), jnp.float32)
    for t in range(7):
        x = x_refs[t][...]
        r = jax.nn.sigmoid(
            jnp.dot(x, ws_ref[0], preferred_element_type=jnp.float32)
            + jnp.dot(h, ws_ref[3], preferred_element_type=jnp.float32)
            + bs_ref[...][0][None, :])
        z = jax.nn.sigmoid(
            jnp.dot(x, ws_ref[1], preferred_element_type=jnp.float32)
            + jnp.dot(h, ws_ref[4], preferred_element_type=jnp.float32)
            + bs_ref[...][1][None, :])
        hn = (jnp.dot(h, ws_ref[5], preferred_element_type=jnp.float32)
              + bs_ref[...][3][None, :])
        inn = (jnp.dot(x, ws_ref[2], preferred_element_type=jnp.float32)
               + bs_ref[...][2][None, :])
        n = jnp.tanh(inn + r * hn)
        h = (1.0 - z) * n + z * h
    emb = h * bn_ref[...][0][None, :] + bn_ref[...][1][None, :]
    emb_ref[...] = emb
    ab_ref[...] = jnp.dot(emb, wa_ref[...], preferred_element_type=jnp.float32)


def _gru(xs, ws, bs, bn2, wa128):
    return pl.pallas_call(
        _gru_body,
        grid=(GRID_N,),
        in_specs=[pl.BlockSpec((ROW_TILE, NGRU), lambda i: (i, 0))
                  for _ in range(7)] + [
            pl.BlockSpec((6, NGRU, NGRU), lambda i: (0, 0, 0)),
            pl.BlockSpec((4, NGRU), lambda i: (0, 0)),
            pl.BlockSpec((2, NGRU), lambda i: (0, 0)),
            pl.BlockSpec((NGRU, 128), lambda i: (0, 0)),
        ],
        out_specs=[
            pl.BlockSpec((ROW_TILE, NGRU), lambda i: (i, 0)),
            pl.BlockSpec((ROW_TILE, 128), lambda i: (i, 0)),
        ],
        out_shape=[
            jax.ShapeDtypeStruct((N_PAD, NGRU), jnp.float32),
            jax.ShapeDtypeStruct((N_PAD, 128), jnp.float32),
        ],
    )(*xs, ws, bs, bn2, wa128)


def _head_body(emb_ref, ngh_ref, wp1a_ref, wp1b_ref, bp1_ref, wp2_ref,
               bp2_ref, out_ref):
    hid = jnp.maximum(
        jnp.dot(emb_ref[...], wp1a_ref[...], preferred_element_type=jnp.float32)
        + jnp.dot(ngh_ref[...], wp1b_ref[...], preferred_element_type=jnp.float32)
        + bp1_ref[...][0],
        0.0,
    )
    logits = (jnp.dot(hid, wp2_ref[...], preferred_element_type=jnp.float32)
              + bp2_ref[...][0])
    m = jnp.max(logits, axis=1, keepdims=True)
    lse = m + jnp.log(jnp.sum(jnp.exp(logits - m), axis=1, keepdims=True))
    out_ref[...] = logits - lse


def _head(emb, ngh, wp1a, wp1b, bp1, wp2pad, bp2pad):
    return pl.pallas_call(
        _head_body,
        grid=(GRID_N,),
        in_specs=[
            pl.BlockSpec((ROW_TILE, NGRU), lambda i: (i, 0)),
            pl.BlockSpec((ROW_TILE, NGRU), lambda i: (i, 0)),
            pl.BlockSpec((NGRU, NHN), lambda i: (0, 0)),
            pl.BlockSpec((NGRU, NHN), lambda i: (0, 0)),
            pl.BlockSpec((1, NHN), lambda i: (0, 0)),
            pl.BlockSpec((NHN, 128), lambda i: (0, 0)),
            pl.BlockSpec((1, 128), lambda i: (0, 0)),
        ],
        out_specs=pl.BlockSpec((ROW_TILE, 128), lambda i: (i, 0)),
        out_shape=jax.ShapeDtypeStruct((N_PAD, 128), jnp.float32),
    )(emb, ngh, wp1a, wp1b, bp1.reshape(1, NHN), wp2pad, bp2pad)


def kernel(adjs, start_day, end_day, W0, b0, W1, b1, W_ih, W_hh, b_ih, b_hh,
           bn_g, bn_b, Wa, ba, Wp1, bp1, Wp2, bp2):
    npad = E_PAD - E
    it = jnp.arange(npad, dtype=jnp.int32)
    pad_rows = N + (it % (N_PAD - N))
    pad_cols = (it * 9973) % N
    pad2 = jnp.stack([pad_rows, pad_cols])
    adjp = jnp.concatenate(
        [adjs.astype(jnp.int32),
         jnp.broadcast_to(pad2[None], (adjs.shape[0], 2, npad))], axis=2)
    adjp = adjp.reshape(adjs.shape[0], 2, EROWS, 128)

    W0p = jnp.zeros((N_PAD, NHID), jnp.float32).at[:N].set(W0)
    W0f = W0p.reshape(N_PAD * 6, 16)

    outs = []
    for i in range(7):
        e = lax.dynamic_index_in_dim(adjp, start_day + i, 0, keepdims=False)
        rows, cols = e[0], e[1]
        x0 = _spmm6(W0f, rows, cols)
        h1 = _layer_a(x0, b0, W1)
        x1 = _spmm4(h1.reshape(N_PAD * 4, 16), rows, cols)
        outs.append(x1)

    b_ih2 = b_ih + W_ih @ b1
    ws = jnp.stack([W_ih[0:64].T, W_ih[64:128].T, W_ih[128:192].T,
                    W_hh[0:64].T, W_hh[64:128].T, W_hh[128:192].T])
    bs = jnp.stack([b_ih2[0:64] + b_hh[0:64],
                    b_ih2[64:128] + b_hh[64:128],
                    b_ih2[128:192],
                    b_hh[128:192]])
    bn2 = jnp.stack([bn_g / jnp.sqrt(1.0 + 1e-5), bn_b])
    wa128 = jnp.zeros((NGRU, 128), jnp.float32)
    wa128 = wa128.at[:, 0].set(Wa[0, :64]).at[:, 1].set(Wa[0, 64:])

    emb, ab = _gru(outs, ws, bs, bn2, wa128)
    aT = ab[:, 0] + ba[0]
    bT = ab[:, 1]

    ea = lax.dynamic_index_in_dim(adjp, end_day + 1, 0, keepdims=False)
    neigh2, deg = _attn(emb.reshape(N_PAD * 2, 32), ea[0], ea[1], aT, bT)
    diag = jnp.where(deg != 0, 1.0 / jnp.where(deg != 0, deg, 1.0), 1.0)
    neigh = neigh2 * diag[:, None]

    wp2pad = jnp.zeros((NHN, 128), jnp.float32).at[:, :2].set(Wp2.T)
    bp2pad = jnp.full((1, 128), -1e30, jnp.float32).at[0, :2].set(bp2)
    pred = _head(emb, neigh, Wp1[:, :64].T, Wp1[:, 64:].T, bp1, wp2pad, bp2pad)
    return pred[:N, :2]


# speedup vs baseline: 1.0710x; 1.0021x over previous
import functools

import jax
import jax.numpy as jnp
from jax import lax
from jax.experimental import pallas as pl
from jax.experimental.pallas import tpu as pltpu
from jax.experimental.pallas import tpu_sc as plsc

N = 50000
E = 800000
NHID = 96
NOUT = 64
NGRU = 64
NHN = 64

N_PAD = 50176
NSLICE = N_PAD // 16
E_PAD = 819200
EROWS = E_PAD // 128
ROWS_PER_TILE = EROWS // 16
ROW_TILE = 512
GRID_N = N_PAD // ROW_TILE

_mesh = lambda: plsc.VectorSubcoreMesh(core_axis_name="c", subcore_axis_name="s")


def _make_spmm(C, day_stride):
    NB = 50

    def body(xf, rows_h, cols_h, out_h,
             colA, rowA, cvA, valA, colB, rowB, cvB, valB, zv, acc,
             sgA, sgB, ssA, ssB):
        cid = lax.axis_index("c")
        sid = lax.axis_index("s")

        def zinit(i, _):
            zv[i, :] = jnp.zeros((16,), jnp.float32)
            return 0

        lax.fori_loop(0, 392, zinit, 0)
        for i in range(8):
            pltpu.sync_copy(zv, acc.at[pl.ds(sid * NSLICE + i * 392, 392)])
        plsc.subcore_barrier()

        def mkhalf(colv, rowv, cv2, val, sg, ss, c):
            off = c

            def load_idx(b):
                r0 = sid * ROWS_PER_TILE + b * 8
                pltpu.sync_copy(cols_h.at[pl.ds(r0, 8)], colv)
                pltpu.sync_copy(rows_h.at[pl.ds(r0, 8)], rowv)
                for j in range(8):
                    for k in range(8):
                        s = colv[j, pl.ds(k * 16, 16)]
                        cv2[j, pl.ds(k * 16, 16)] = s * C + off

            def fire_g():
                for j in range(8):
                    pltpu.async_copy(xf.at[cv2.at[j]], val.at[j], sg)

            def wait_g():
                for j in range(8):
                    pltpu.make_async_copy(xf.at[cv2.at[j]], val.at[j],
                                          sg).wait()

            def fire_s():
                for j in range(8):
                    pltpu.async_copy(val.at[j], acc.at[rowv.at[j]], ss,
                                     add=True)

            def wait_s():
                for j in range(8):
                    pltpu.make_async_copy(val.at[j], acc.at[rowv.at[j]],
                                          ss).wait()

            return load_idx, fire_g, wait_g, fire_s, wait_s

        for c in range(C):
            @pl.when(cid == c % 2)
            def _(c=c):
                liA, fgA, wgA, fsA, wsA = mkhalf(colA, rowA, cvA, valA,
                                                 sgA, ssA, c)
                liB, fgB, wgB, fsB, wsB = mkhalf(colB, rowB, cvB, valB,
                                                 sgB, ssB, c)
                liA(0)
                fgA()

                def blk(i, _):
                    @pl.when(i > 0)
                    def _w():
                        wsB()

                    liB(2 * i + 1)
                    fgB()
                    wgA()
                    fsA()
                    wsA()

                    @pl.when(i < NB // 2 - 1)
                    def _p():
                        liA(2 * i + 2)
                        fgA()

                    wgB()
                    fsB()
                    return 0

                lax.fori_loop(0, NB // 2, blk, 0)
                wsB()
                plsc.subcore_barrier()
                pltpu.sync_copy(acc.at[pl.ds(sid * NSLICE, NSLICE)],
                                out_h.at[pl.ds(sid * NSLICE, NSLICE),
                                         pl.ds(c * 16, 16)])
                for i in range(8):
                    pltpu.sync_copy(
                        zv, acc.at[pl.ds(sid * NSLICE + i * 392, 392)])
                plsc.subcore_barrier()

    return pl.kernel(
        body,
        out_type=jax.ShapeDtypeStruct((N_PAD, C * 16), jnp.float32),
        mesh=_mesh(),
        compiler_params=pltpu.CompilerParams(use_tc_tiling_on_sc=False),
        scratch_types=[
            pltpu.VMEM((8, 128), jnp.int32),
            pltpu.VMEM((8, 128), jnp.int32),
            pltpu.VMEM((8, 128), jnp.int32),
            pltpu.VMEM((8, 128, 16), jnp.float32),
            pltpu.VMEM((8, 128), jnp.int32),
            pltpu.VMEM((8, 128), jnp.int32),
            pltpu.VMEM((8, 128), jnp.int32),
            pltpu.VMEM((8, 128, 16), jnp.float32),
            pltpu.VMEM((392, 16), jnp.float32),
            pltpu.VMEM_SHARED((N_PAD, 16), jnp.float32),
            pltpu.SemaphoreType.DMA,
            pltpu.SemaphoreType.DMA,
            pltpu.SemaphoreType.DMA,
            pltpu.SemaphoreType.DMA,
        ],
    )


_spmm6 = _make_spmm(6, 0)
_spmm4 = _make_spmm(4, 0)


def _attn_body(embf, rows_h, cols_h, aT_h, bT_h, out_h, deg_h,
               colA, rowA, cvA, avA, bvA, wvA, mvA, ebA,
               colB, rowB, cvB, avB, bvB, wvB, mvB, ebB,
               zv, zv1, aT_s, bT_s, acc, dacc,
               sgA, sgB, stA, stB, ssA, ssB, sdA, sdB):
    cid = lax.axis_index("c")
    sid = lax.axis_index("s")

    @pl.when(sid == 0)
    def _():
        pltpu.sync_copy(aT_h, aT_s)
        pltpu.sync_copy(bT_h, bT_s)

    def zinit(i, _):
        zv[i, pl.ds(0, 16)] = jnp.zeros((16,), jnp.float32)
        zv[i, pl.ds(16, 16)] = jnp.zeros((16,), jnp.float32)
        return 0

    lax.fori_loop(0, 16, zinit, 0)

    def zinit1(i, _):
        zv1[pl.ds(i * 16, 16)] = jnp.zeros((16,), jnp.float32)
        return 0

    lax.fori_loop(0, 28, zinit1, 0)
    for i in range(196):
        pltpu.sync_copy(zv, acc.at[pl.ds(sid * NSLICE + i * 16, 16)])
    for i in range(7):
        pltpu.sync_copy(zv1, dacc.at[pl.ds(sid * NSLICE + i * 448, 448)])
    plsc.subcore_barrier()

    for c in range(2):
        @pl.when(cid == c)
        def _(c=c):
            def mkhalf(colv, rowv, cv2, av, bv, wv, mv, ebuf, sg, st, ss, sd):
                def li(b):
                    r0 = sid * ROWS_PER_TILE + b * 2
                    pltpu.sync_copy(cols_h.at[pl.ds(r0, 2)], colv)
                    pltpu.sync_copy(rows_h.at[pl.ds(r0, 2)], rowv)
                    for j in range(2):
                        for k in range(8):
                            s = colv[j, pl.ds(k * 16, 16)]
                            cv2[j, pl.ds(k * 16, 16)] = s * 2 + c

                def fg():
                    for j in range(2):
                        pltpu.async_copy(aT_s.at[rowv.at[j]], av.at[j], st)
                        pltpu.async_copy(bT_s.at[colv.at[j]], bv.at[j], st)
                        pltpu.async_copy(embf.at[cv2.at[j]], ebuf.at[j], sg)

                def wg():
                    for j in range(2):
                        pltpu.make_async_copy(aT_s.at[rowv.at[j]], av.at[j],
                                              st).wait()
                        pltpu.make_async_copy(bT_s.at[colv.at[j]], bv.at[j],
                                              st).wait()
                        pltpu.make_async_copy(embf.at[cv2.at[j]], ebuf.at[j],
                                              sg).wait()

                def compute():
                    for j in range(2):
                        for k in range(8):
                            r16 = rowv[j, pl.ds(k * 16, 16)]
                            c16 = colv[j, pl.ds(k * 16, 16)]
                            a16 = av[j, pl.ds(k * 16, 16)]
                            b16 = bv[j, pl.ds(k * 16, 16)]
                            w = 1.0 / (1.0 + jnp.exp(-(a16 + b16)))
                            m = r16 != c16
                            w = jnp.where(m, w, jnp.zeros((16,), jnp.float32))
                            wv[j, pl.ds(k * 16, 16)] = w
                            if c == 0:
                                mv[j, pl.ds(k * 16, 16)] = jnp.where(
                                    m, jnp.ones((16,), jnp.float32),
                                    jnp.zeros((16,), jnp.float32))

                    def scale(kk, _):
                        for j in range(2):
                            w16 = wv[j, pl.ds(kk * 16, 16)]
                            for l in range(16):
                                s16 = jnp.full((16,), w16[l], jnp.float32)
                                t0 = ebuf[j, kk * 16 + l, pl.ds(0, 16)]
                                ebuf[j, kk * 16 + l, pl.ds(0, 16)] = t0 * s16
                                t1 = ebuf[j, kk * 16 + l, pl.ds(16, 16)]
                                ebuf[j, kk * 16 + l, pl.ds(16, 16)] = t1 * s16
                        return 0

                    lax.fori_loop(0, 8, scale, 0)

                def fs():
                    for j in range(2):
                        pltpu.async_copy(ebuf.at[j], acc.at[rowv.at[j]], ss,
                                         add=True)
                        if c == 0:
                            pltpu.async_copy(mv.at[j], dacc.at[rowv.at[j]],
                                             sd, add=True)

                def ws():
                    for j in range(2):
                        pltpu.make_async_copy(ebuf.at[j], acc.at[rowv.at[j]],
                                              ss).wait()
                        if c == 0:
                            pltpu.make_async_copy(mv.at[j],
                                                  dacc.at[rowv.at[j]],
                                                  sd).wait()

                return li, fg, wg, compute, fs, ws

            liA, fgA, wgA, cpA, fsA, wsA = mkhalf(colA, rowA, cvA, avA, bvA,
                                                  wvA, mvA, ebA, sgA, stA,
                                                  ssA, sdA)
            liB, fgB, wgB, cpB, fsB, wsB = mkhalf(colB, rowB, cvB, avB, bvB,
                                                  wvB, mvB, ebB, sgB, stB,
                                                  ssB, sdB)
            liA(0)
            fgA()

            def blk(i, _):
                @pl.when(i > 0)
                def _w():
                    wsB()

                liB(2 * i + 1)
                fgB()
                wgA()
                cpA()
                fsA()
                wsA()

                @pl.when(i < 99)
                def _p():
                    liA(2 * i + 2)
                    fgA()

                wgB()
                cpB()
                fsB()
                return 0

            lax.fori_loop(0, 100, blk, 0)
            wsB()
            plsc.subcore_barrier()
            pltpu.sync_copy(acc.at[pl.ds(sid * NSLICE, NSLICE)],
                            out_h.at[pl.ds(sid * NSLICE, NSLICE),
                                     pl.ds(c * 32, 32)])
            if c == 0:
                pltpu.sync_copy(dacc.at[pl.ds(sid * NSLICE, NSLICE)],
                                deg_h.at[pl.ds(sid * NSLICE, NSLICE)])


_attn = pl.kernel(
    _attn_body,
    out_type=(jax.ShapeDtypeStruct((N_PAD, NOUT), jnp.float32),
              jax.ShapeDtypeStruct((N_PAD,), jnp.float32)),
    mesh=_mesh(),
    compiler_params=pltpu.CompilerParams(use_tc_tiling_on_sc=False),
    scratch_types=(
        [pltpu.VMEM((2, 128), jnp.int32)] * 3
        + [pltpu.VMEM((2, 128), jnp.float32)] * 4
        + [pltpu.VMEM((2, 128, 32), jnp.float32)]
        + [pltpu.VMEM((2, 128), jnp.int32)] * 3
        + [pltpu.VMEM((2, 128), jnp.float32)] * 4
        + [pltpu.VMEM((2, 128, 32), jnp.float32)]
        + [
            pltpu.VMEM((16, 32), jnp.float32),
            pltpu.VMEM((448,), jnp.float32),
            pltpu.VMEM_SHARED((N_PAD,), jnp.float32),
            pltpu.VMEM_SHARED((N_PAD,), jnp.float32),
            pltpu.VMEM_SHARED((N_PAD, 32), jnp.float32),
            pltpu.VMEM_SHARED((N_PAD,), jnp.float32),
        ]
        + [pltpu.SemaphoreType.DMA] * 8
    ),
)


def _layer_a_body(x_ref, b0_ref, w1_ref, o_ref):
    x = jnp.maximum(x_ref[...] + b0_ref[...][0], 0.0)
    o_ref[...] = jnp.dot(x, w1_ref[...], preferred_element_type=jnp.float32)


def _layer_a(x, b0, W1):
    rows = x.shape[0]
    return pl.pallas_call(
        _layer_a_body,
        grid=(rows // ROW_TILE,),
        in_specs=[
            pl.BlockSpec((ROW_TILE, NHID), lambda i: (i, 0)),
            pl.BlockSpec((1, NHID), lambda i: (0, 0)),
            pl.BlockSpec((NHID, NOUT), lambda i: (0, 0)),
        ],
        out_specs=pl.BlockSpec((ROW_TILE, NOUT), lambda i: (i, 0)),
        out_shape=jax.ShapeDtypeStruct((rows, NOUT), jnp.float32),
    )(x, b0.reshape(1, NHID), W1)


def _gru_body(*refs):
    x_refs = refs[:7]
    ws_ref, bs_ref, bn_ref, wa_ref, emb_ref, ab_ref = refs[7:]
    h = jnp.zeros((ROW_TILE, NGRU), jnp.float32)
    for t in range(7):
        x = x_refs[t][...]
        r = jax.nn.sigmoid(
            jnp.dot(x, ws_ref[0], preferred_element_type=jnp.float32)
            + jnp.dot(h, ws_ref[3], preferred_element_type=jnp.float32)
            + bs_ref[...][0][None, :])
        z = jax.nn.sigmoid(
            jnp.dot(x, ws_ref[1], preferred_element_type=jnp.float32)
            + jnp.dot(h, ws_ref[4], preferred_element_type=jnp.float32)
            + bs_ref[...][1][None, :])
        hn = (jnp.dot(h, ws_ref[5], preferred_element_type=jnp.float32)
              + bs_ref[...][3][None, :])
        inn = (jnp.dot(x, ws_ref[2], preferred_element_type=jnp.float32)
               + bs_ref[...][2][None, :])
        n = jnp.tanh(inn + r * hn)
        h = (1.0 - z) * n + z * h
    emb = h * bn_ref[...][0][None, :] + bn_ref[...][1][None, :]
    emb_ref[...] = emb
    ab_ref[...] = jnp.dot(emb, wa_ref[...], preferred_element_type=jnp.float32)


def _gru(xs, ws, bs, bn2, wa128):
    return pl.pallas_call(
        _gru_body,
        grid=(GRID_N,),
        in_specs=[pl.BlockSpec((ROW_TILE, NGRU), lambda i: (i, 0))
                  for _ in range(7)] + [
            pl.BlockSpec((6, NGRU, NGRU), lambda i: (0, 0, 0)),
            pl.BlockSpec((4, NGRU), lambda i: (0, 0)),
            pl.BlockSpec((2, NGRU), lambda i: (0, 0)),
            pl.BlockSpec((NGRU, 128), lambda i: (0, 0)),
        ],
        out_specs=[
            pl.BlockSpec((ROW_TILE, NGRU), lambda i: (i, 0)),
            pl.BlockSpec((ROW_TILE, 128), lambda i: (i, 0)),
        ],
        out_shape=[
            jax.ShapeDtypeStruct((N_PAD, NGRU), jnp.float32),
            jax.ShapeDtypeStruct((N_PAD, 128), jnp.float32),
        ],
    )(*xs, ws, bs, bn2, wa128)


def _head_body(emb_ref, ngh_ref, wp1a_ref, wp1b_ref, bp1_ref, wp2_ref,
               bp2_ref, out_ref):
    hid = jnp.maximum(
        jnp.dot(emb_ref[...], wp1a_ref[...], preferred_element_type=jnp.float32)
        + jnp.dot(ngh_ref[...], wp1b_ref[...], preferred_element_type=jnp.float32)
        + bp1_ref[...][0],
        0.0,
    )
    logits = (jnp.dot(hid, wp2_ref[...], preferred_element_type=jnp.float32)
              + bp2_ref[...][0])
    m = jnp.max(logits, axis=1, keepdims=True)
    lse = m + jnp.log(jnp.sum(jnp.exp(logits - m), axis=1, keepdims=True))
    out_ref[...] = logits - lse


def _head(emb, ngh, wp1a, wp1b, bp1, wp2pad, bp2pad):
    return pl.pallas_call(
        _head_body,
        grid=(GRID_N,),
        in_specs=[
            pl.BlockSpec((ROW_TILE, NGRU), lambda i: (i, 0)),
            pl.BlockSpec((ROW_TILE, NGRU), lambda i: (i, 0)),
            pl.BlockSpec((NGRU, NHN), lambda i: (0, 0)),
            pl.BlockSpec((NGRU, NHN), lambda i: (0, 0)),
            pl.BlockSpec((1, NHN), lambda i: (0, 0)),
            pl.BlockSpec((NHN, 128), lambda i: (0, 0)),
            pl.BlockSpec((1, 128), lambda i: (0, 0)),
        ],
        out_specs=pl.BlockSpec((ROW_TILE, 128), lambda i: (i, 0)),
        out_shape=jax.ShapeDtypeStruct((N_PAD, 128), jnp.float32),
    )(emb, ngh, wp1a, wp1b, bp1.reshape(1, NHN), wp2pad, bp2pad)


def kernel(adjs, start_day, end_day, W0, b0, W1, b1, W_ih, W_hh, b_ih, b_hh,
           bn_g, bn_b, Wa, ba, Wp1, bp1, Wp2, bp2):
    npad = E_PAD - E
    it = jnp.arange(npad, dtype=jnp.int32)
    pad_rows = N + (it % (N_PAD - N))
    pad_cols = (it * 9973) % N
    pad2 = jnp.stack([pad_rows, pad_cols])
    adjp = jnp.concatenate(
        [adjs.astype(jnp.int32),
         jnp.broadcast_to(pad2[None], (adjs.shape[0], 2, npad))], axis=2)
    adjp = adjp.reshape(adjs.shape[0], 2, EROWS, 128)

    W0f = W0.reshape(N * 6, 16)

    outs = []
    for i in range(7):
        e = lax.dynamic_index_in_dim(adjp, start_day + i, 0, keepdims=False)
        rows, cols = e[0], e[1]
        x0 = _spmm6(W0f, rows, cols)
        h1 = _layer_a(x0, b0, W1)
        x1 = _spmm4(h1.reshape(N_PAD * 4, 16), rows, cols)
        outs.append(x1)

    b_ih2 = b_ih + W_ih @ b1
    ws = jnp.stack([W_ih[0:64].T, W_ih[64:128].T, W_ih[128:192].T,
                    W_hh[0:64].T, W_hh[64:128].T, W_hh[128:192].T])
    bs = jnp.stack([b_ih2[0:64] + b_hh[0:64],
                    b_ih2[64:128] + b_hh[64:128],
                    b_ih2[128:192],
                    b_hh[128:192]])
    bn2 = jnp.stack([bn_g / jnp.sqrt(1.0 + 1e-5), bn_b])
    wa128 = jnp.zeros((NGRU, 128), jnp.float32)
    wa128 = wa128.at[:, 0].set(Wa[0, :64]).at[:, 1].set(Wa[0, 64:])

    emb, ab = _gru(outs, ws, bs, bn2, wa128)
    aT = ab[:, 0] + ba[0]
    bT = ab[:, 1]

    ea = lax.dynamic_index_in_dim(adjp, end_day + 1, 0, keepdims=False)
    neigh2, deg = _attn(emb.reshape(N_PAD * 2, 32), ea[0], ea[1], aT, bT)
    diag = jnp.where(deg != 0, 1.0 / jnp.where(deg != 0, deg, 1.0), 1.0)
    neigh = neigh2 * diag[:, None]

    wp2pad = jnp.zeros((NHN, 128), jnp.float32).at[:, :2].set(Wp2.T)
    bp2pad = jnp.full((1, 128), -1e30, jnp.float32).at[0, :2].set(bp2)
    pred = _head(emb, neigh, Wp1[:, :64].T, Wp1[:, 64:].T, bp1, wp2pad, bp2pad)
    return pred[:N, :2]
